# Initial kernel scaffold; baseline (speedup 1.0000x reference)
import jax
import jax.numpy as jnp
from jax import lax
from jax.experimental import pallas as pl
from jax.experimental.pallas import tpu as pltpu

_N = 20000
_MAX_OUT = 300
_ROWS = 160
_COLS = 128
_PAD = _ROWS * _COLS - _N
_NEG = jnp.float32(-1e9)
_PAD_SCORE = jnp.float32(-3.0e38)
_THRESH = jnp.float32(0.7)


def _nms_body(x1_ref, y1_ref, x2_ref, y2_ref, s_ref, out_ref, sw_ref, area_ref):
    sw_ref[:] = s_ref[:]
    area_ref[:] = (x2_ref[:] - x1_ref[:]) * (y2_ref[:] - y1_ref[:])
    out_ref[:] = jnp.zeros((8, 304), jnp.float32)

    row_iota = lax.broadcasted_iota(jnp.int32, (_ROWS, _COLS), 0)
    col_iota = lax.broadcasted_iota(jnp.int32, (_ROWS, _COLS), 1)
    lin = row_iota * _COLS + col_iota

    orow = lax.broadcasted_iota(jnp.int32, (8, 304), 0)
    ocol = lax.broadcasted_iota(jnp.int32, (8, 304), 1)

    def body(t, carry):
        s = sw_ref[:]
        m = jnp.max(s)
        idx = jnp.min(jnp.where(s == m, lin, jnp.int32(2**31 - 1)))
        r = idx // _COLS
        c = lax.rem(idx, _COLS)
        sx1 = x1_ref[r, c]
        sy1 = y1_ref[r, c]
        sx2 = x2_ref[r, c]
        sy2 = y2_ref[r, c]
        sarea = (sx2 - sx1) * (sy2 - sy1)

        xx1 = jnp.maximum(sx1, x1_ref[:])
        yy1 = jnp.maximum(sy1, y1_ref[:])
        xx2 = jnp.minimum(sx2, x2_ref[:])
        yy2 = jnp.minimum(sy2, y2_ref[:])
        inter = jnp.maximum(xx2 - xx1, 0.0) * jnp.maximum(yy2 - yy1, 0.0)
        iou = inter / (sarea + area_ref[:] - inter + jnp.float32(1e-9))
        sw_ref[:] = jnp.where(iou > _THRESH, _NEG, s)

        v = jnp.where(
            orow == 0,
            sx1,
            jnp.where(
                orow == 1,
                sy1,
                jnp.where(orow == 2, sx2, jnp.where(orow == 3, sy2, m)),
            ),
        )
        out_ref[:] = jnp.where(ocol == t, v, out_ref[:])
        return carry

    lax.fori_loop(0, _MAX_OUT, body, 0)


def _run_nms(x1, y1, x2, y2, s):
    return pl.pallas_call(
        _nms_body,
        out_shape=jax.ShapeDtypeStruct((8, 304), jnp.float32),
        in_specs=[pl.BlockSpec(memory_space=pltpu.VMEM)] * 5,
        out_specs=pl.BlockSpec(memory_space=pltpu.VMEM),
        scratch_shapes=[
            pltpu.VMEM((_ROWS, _COLS), jnp.float32),
            pltpu.VMEM((_ROWS, _COLS), jnp.float32),
        ],
    )(x1, y1, x2, y2, s)


def kernel(boxes, scores):
    zpad = jnp.zeros((_PAD,), jnp.float32)
    x1 = jnp.concatenate([boxes[:, 0], zpad]).reshape(_ROWS, _COLS)
    y1 = jnp.concatenate([boxes[:, 1], zpad]).reshape(_ROWS, _COLS)
    x2 = jnp.concatenate([boxes[:, 2], zpad]).reshape(_ROWS, _COLS)
    y2 = jnp.concatenate([boxes[:, 3], zpad]).reshape(_ROWS, _COLS)
    s = jnp.concatenate([scores, jnp.full((_PAD,), _PAD_SCORE)]).reshape(
        _ROWS, _COLS
    )
    out = _run_nms(x1, y1, x2, y2, s)
    return out[:5, :_MAX_OUT].T


# speedup vs baseline: 18.9997x; 18.9997x over previous
import jax
import jax.numpy as jnp
from jax import lax
from jax.experimental import pallas as pl
from jax.experimental.pallas import tpu as pltpu

_N = 20000
_MAX_OUT = 300
_ROWS = 160
_COLS = 128
_PAD = _ROWS * _COLS - _N
_NEG = -1e9
_PAD_SCORE = -3.0e38
_THRESH = 0.7


def _nms_body(x1_ref, y1_ref, x2_ref, y2_ref, s_ref, out_ref, sw_ref, area_ref):
    sw_ref[:] = s_ref[:]
    area_ref[:] = (x2_ref[:] - x1_ref[:]) * (y2_ref[:] - y1_ref[:])
    out_ref[:] = jnp.zeros((8, 304), jnp.float32)

    row_iota = lax.broadcasted_iota(jnp.int32, (_ROWS, _COLS), 0)
    col_iota = lax.broadcasted_iota(jnp.int32, (_ROWS, _COLS), 1)
    lin = row_iota * _COLS + col_iota

    orow = lax.broadcasted_iota(jnp.int32, (8, 304), 0)
    ocol = lax.broadcasted_iota(jnp.int32, (8, 304), 1)

    def body(t, carry):
        s = sw_ref[:]
        m = jnp.max(s)
        idx = jnp.min(jnp.where(s == m, lin, jnp.int32(2**31 - 1)))
        r = idx // _COLS
        c = lax.rem(idx, _COLS)
        lane = lax.broadcasted_iota(jnp.int32, (1, _COLS), 1)
        sel = lane == c

        def pick(ref):
            return jnp.sum(jnp.where(sel, ref[pl.ds(r, 1), :], 0.0))

        sx1 = pick(x1_ref)
        sy1 = pick(y1_ref)
        sx2 = pick(x2_ref)
        sy2 = pick(y2_ref)
        sarea = (sx2 - sx1) * (sy2 - sy1)

        xx1 = jnp.maximum(sx1, x1_ref[:])
        yy1 = jnp.maximum(sy1, y1_ref[:])
        xx2 = jnp.minimum(sx2, x2_ref[:])
        yy2 = jnp.minimum(sy2, y2_ref[:])
        inter = jnp.maximum(xx2 - xx1, 0.0) * jnp.maximum(yy2 - yy1, 0.0)
        iou = inter / (sarea + area_ref[:] - inter + jnp.float32(1e-9))
        sw_ref[:] = jnp.where(iou > jnp.float32(_THRESH), jnp.float32(_NEG), s)

        v = jnp.where(
            orow == 0,
            sx1,
            jnp.where(
                orow == 1,
                sy1,
                jnp.where(orow == 2, sx2, jnp.where(orow == 3, sy2, m)),
            ),
        )
        out_ref[:] = jnp.where(ocol == t, v, out_ref[:])
        return carry

    lax.fori_loop(0, _MAX_OUT, body, 0)


def _run_nms(x1, y1, x2, y2, s):
    return pl.pallas_call(
        _nms_body,
        out_shape=jax.ShapeDtypeStruct((8, 304), jnp.float32),
        in_specs=[pl.BlockSpec(memory_space=pltpu.VMEM)] * 5,
        out_specs=pl.BlockSpec(memory_space=pltpu.VMEM),
        scratch_shapes=[
            pltpu.VMEM((_ROWS, _COLS), jnp.float32),
            pltpu.VMEM((_ROWS, _COLS), jnp.float32),
        ],
    )(x1, y1, x2, y2, s)


def kernel(boxes, scores):
    zpad = jnp.zeros((_PAD,), jnp.float32)
    x1 = jnp.concatenate([boxes[:, 0], zpad]).reshape(_ROWS, _COLS)
    y1 = jnp.concatenate([boxes[:, 1], zpad]).reshape(_ROWS, _COLS)
    x2 = jnp.concatenate([boxes[:, 2], zpad]).reshape(_ROWS, _COLS)
    y2 = jnp.concatenate([boxes[:, 3], zpad]).reshape(_ROWS, _COLS)
    s = jnp.concatenate([scores, jnp.full((_PAD,), _PAD_SCORE)]).reshape(
        _ROWS, _COLS
    )
    out = _run_nms(x1, y1, x2, y2, s)
    return out[:5, :_MAX_OUT].T


# speedup vs baseline: 28.9612x; 1.5243x over previous
import jax
import jax.numpy as jnp
from jax import lax
from jax.experimental import pallas as pl
from jax.experimental.pallas import tpu as pltpu

_N = 20000
_MAX_OUT = 300
_ROWS = 160
_COLS = 128
_PAD = _ROWS * _COLS - _N
_NEG = -1e9
_PAD_SCORE = -3.0e38
_THRESH = 0.7


def _nms_body(
    x1_ref, y1_ref, x2_ref, y2_ref, s_ref,
    x1s_ref, y1s_ref, x2s_ref, y2s_ref,
    out_ref, area_ref,
):
    area_ref[:] = (x2_ref[:] - x1_ref[:]) * (y2_ref[:] - y1_ref[:])

    row_iota = lax.broadcasted_iota(jnp.int32, (_ROWS, _COLS), 0)
    col_iota = lax.broadcasted_iota(jnp.int32, (_ROWS, _COLS), 1)
    linf = (row_iota * _COLS + col_iota).astype(jnp.float32)
    lane = lax.broadcasted_iota(jnp.int32, (1, _COLS), 1)

    def body(t, s):
        m = jnp.max(s)
        idx = jnp.min(
            jnp.where(s == m, linf, jnp.float32(3.0e38))
        ).astype(jnp.int32)

        sx1 = x1s_ref[idx]
        sy1 = y1s_ref[idx]
        sx2 = x2s_ref[idx]
        sy2 = y2s_ref[idx]
        sarea = (sx2 - sx1) * (sy2 - sy1)

        xx1 = jnp.maximum(sx1, x1_ref[:])
        yy1 = jnp.maximum(sy1, y1_ref[:])
        xx2 = jnp.minimum(sx2, x2_ref[:])
        yy2 = jnp.minimum(sy2, y2_ref[:])
        inter = jnp.maximum(xx2 - xx1, 0.0) * jnp.maximum(yy2 - yy1, 0.0)
        iou = inter / (sarea + area_ref[:] - inter + jnp.float32(1e-9))
        s_new = jnp.where(iou > jnp.float32(_THRESH), jnp.float32(_NEG), s)

        detrow = jnp.where(
            lane == 0,
            sx1,
            jnp.where(
                lane == 1,
                sy1,
                jnp.where(lane == 2, sx2, jnp.where(lane == 3, sy2, m)),
            ),
        )
        out_ref[pl.ds(t, 1), :] = detrow
        return s_new

    lax.fori_loop(0, _MAX_OUT, body, s_ref[:])


def _run_nms(x1, y1, x2, y2, s, x1f, y1f, x2f, y2f):
    return pl.pallas_call(
        _nms_body,
        out_shape=jax.ShapeDtypeStruct((_MAX_OUT + 4, _COLS), jnp.float32),
        in_specs=[pl.BlockSpec(memory_space=pltpu.VMEM)] * 5
        + [pl.BlockSpec(memory_space=pltpu.SMEM)] * 4,
        out_specs=pl.BlockSpec(memory_space=pltpu.VMEM),
        scratch_shapes=[
            pltpu.VMEM((_ROWS, _COLS), jnp.float32),
        ],
    )(x1, y1, x2, y2, s, x1f, y1f, x2f, y2f)


def kernel(boxes, scores):
    zpad = jnp.zeros((_PAD,), jnp.float32)
    x1f = jnp.concatenate([boxes[:, 0], zpad])
    y1f = jnp.concatenate([boxes[:, 1], zpad])
    x2f = jnp.concatenate([boxes[:, 2], zpad])
    y2f = jnp.concatenate([boxes[:, 3], zpad])
    x1 = x1f.reshape(_ROWS, _COLS)
    y1 = y1f.reshape(_ROWS, _COLS)
    x2 = x2f.reshape(_ROWS, _COLS)
    y2 = y2f.reshape(_ROWS, _COLS)
    s = jnp.concatenate([scores, jnp.full((_PAD,), _PAD_SCORE)]).reshape(
        _ROWS, _COLS
    )
    out = _run_nms(x1, y1, x2, y2, s, x1f, y1f, x2f, y2f)
    return out[:_MAX_OUT, :5]


# speedup vs baseline: 30.4747x; 1.0523x over previous
import jax
import jax.numpy as jnp
from jax import lax
from jax.experimental import pallas as pl
from jax.experimental.pallas import tpu as pltpu

_N = 20000
_MAX_OUT = 300
_ROWS = 160
_COLS = 128
_PAD = _ROWS * _COLS - _N
_NEG = -1e9
_PAD_SCORE = -3.0e38
_THRESH = 0.7
_NBLK = _ROWS // 8


def _nms_body(
    x1_ref, y1_ref, x2_ref, y2_ref, s_ref,
    x1s_ref, y1s_ref, x2s_ref, y2s_ref,
    out_ref, area_ref,
):
    area_ref[:] = (x2_ref[:] - x1_ref[:]) * (y2_ref[:] - y1_ref[:])

    row_iota = lax.broadcasted_iota(jnp.int32, (_ROWS, _COLS), 0)
    col_iota = lax.broadcasted_iota(jnp.int32, (_ROWS, _COLS), 1)
    linf = (row_iota * _COLS + col_iota).astype(jnp.float32)
    lane = lax.broadcasted_iota(jnp.int32, (1, _COLS), 1)

    def body(t, s):
        def merge(cv, ci, nv, ni):
            take = (nv > cv) | ((nv == cv) & (ni < ci))
            return jnp.where(take, nv, cv), jnp.where(take, ni, ci)

        vals = [s[8 * i : 8 * i + 8] for i in range(_NBLK)]
        idxs = [linf[8 * i : 8 * i + 8] for i in range(_NBLK)]
        while len(vals) > 1:
            nv, ni = [], []
            for i in range(0, len(vals) - 1, 2):
                v, ix = merge(vals[i], idxs[i], vals[i + 1], idxs[i + 1])
                nv.append(v)
                ni.append(ix)
            if len(vals) % 2:
                nv.append(vals[-1])
                ni.append(idxs[-1])
            vals, idxs = nv, ni
        v8, i8 = vals[0], idxs[0]
        for sh in (4, 2, 1):
            vr = pltpu.roll(v8, 8 - sh, 0)
            ir = pltpu.roll(i8, 8 - sh, 0)
            v8, i8 = merge(v8, i8, vr, ir)
        a1v = v8[0:1, :]
        a1i = i8[0:1, :]

        m = jnp.max(a1v, axis=1, keepdims=True)
        idx = jnp.min(
            jnp.where(a1v == m, a1i, jnp.float32(3.0e38))
        ).astype(jnp.int32)

        sx1 = x1s_ref[idx]
        sy1 = y1s_ref[idx]
        sx2 = x2s_ref[idx]
        sy2 = y2s_ref[idx]
        sarea = (sx2 - sx1) * (sy2 - sy1)

        xx1 = jnp.maximum(sx1, x1_ref[:])
        yy1 = jnp.maximum(sy1, y1_ref[:])
        xx2 = jnp.minimum(sx2, x2_ref[:])
        yy2 = jnp.minimum(sy2, y2_ref[:])
        inter = jnp.maximum(xx2 - xx1, 0.0) * jnp.maximum(yy2 - yy1, 0.0)
        iou = inter / (sarea + area_ref[:] - inter + jnp.float32(1e-9))
        s_new = jnp.where(iou > jnp.float32(_THRESH), jnp.float32(_NEG), s)

        detrow = jnp.where(
            lane == 0,
            sx1,
            jnp.where(
                lane == 1,
                sy1,
                jnp.where(lane == 2, sx2, jnp.where(lane == 3, sy2, m)),
            ),
        )
        out_ref[pl.ds(t, 1), :] = detrow
        return s_new

    lax.fori_loop(0, _MAX_OUT, body, s_ref[:])


def _run_nms(x1, y1, x2, y2, s, x1f, y1f, x2f, y2f):
    return pl.pallas_call(
        _nms_body,
        out_shape=jax.ShapeDtypeStruct((_MAX_OUT + 4, _COLS), jnp.float32),
        in_specs=[pl.BlockSpec(memory_space=pltpu.VMEM)] * 5
        + [pl.BlockSpec(memory_space=pltpu.SMEM)] * 4,
        out_specs=pl.BlockSpec(memory_space=pltpu.VMEM),
        scratch_shapes=[
            pltpu.VMEM((_ROWS, _COLS), jnp.float32),
        ],
    )(x1, y1, x2, y2, s, x1f, y1f, x2f, y2f)


def kernel(boxes, scores):
    zpad = jnp.zeros((_PAD,), jnp.float32)
    x1f = jnp.concatenate([boxes[:, 0], zpad])
    y1f = jnp.concatenate([boxes[:, 1], zpad])
    x2f = jnp.concatenate([boxes[:, 2], zpad])
    y2f = jnp.concatenate([boxes[:, 3], zpad])
    x1 = x1f.reshape(_ROWS, _COLS)
    y1 = y1f.reshape(_ROWS, _COLS)
    x2 = x2f.reshape(_ROWS, _COLS)
    y2 = y2f.reshape(_ROWS, _COLS)
    s = jnp.concatenate([scores, jnp.full((_PAD,), _PAD_SCORE)]).reshape(
        _ROWS, _COLS
    )
    out = _run_nms(x1, y1, x2, y2, s, x1f, y1f, x2f, y2f)
    return out[:_MAX_OUT, :5]


# speedup vs baseline: 31.0471x; 1.0188x over previous
import jax
import jax.numpy as jnp
from jax import lax
from jax.experimental import pallas as pl
from jax.experimental.pallas import tpu as pltpu

_N = 20000
_MAX_OUT = 300
_ROWS = 160
_COLS = 128
_PAD = _ROWS * _COLS - _N
_NEG = -1e9
_PAD_SCORE = -3.0e38
_THRESH = 0.7
_NBLK = _ROWS // 8


def _nms_body(
    x1_ref, y1_ref, x2_ref, y2_ref, s_ref,
    x1s_ref, y1s_ref, x2s_ref, y2s_ref,
    out_ref, area_ref,
):
    area_ref[:] = (x2_ref[:] - x1_ref[:]) * (y2_ref[:] - y1_ref[:])

    row_iota = lax.broadcasted_iota(jnp.int32, (_ROWS, _COLS), 0)
    col_iota = lax.broadcasted_iota(jnp.int32, (_ROWS, _COLS), 1)
    linf = (row_iota * _COLS + col_iota).astype(jnp.float32)
    lane = lax.broadcasted_iota(jnp.int32, (1, _COLS), 1)

    def one_round(t, s):
        def merge(cv, ci, nv, ni):
            take = (nv > cv) | ((nv == cv) & (ni < ci))
            return jnp.where(take, nv, cv), jnp.where(take, ni, ci)

        vals = [s[8 * i : 8 * i + 8] for i in range(_NBLK)]
        idxs = [linf[8 * i : 8 * i + 8] for i in range(_NBLK)]
        while len(vals) > 1:
            nv, ni = [], []
            for i in range(0, len(vals) - 1, 2):
                v, ix = merge(vals[i], idxs[i], vals[i + 1], idxs[i + 1])
                nv.append(v)
                ni.append(ix)
            if len(vals) % 2:
                nv.append(vals[-1])
                ni.append(idxs[-1])
            vals, idxs = nv, ni
        v8, i8 = vals[0], idxs[0]
        for sh in (4, 2, 1):
            vr = pltpu.roll(v8, 8 - sh, 0)
            ir = pltpu.roll(i8, 8 - sh, 0)
            v8, i8 = merge(v8, i8, vr, ir)
        a1v = v8[0:1, :]
        a1i = i8[0:1, :]

        m = jnp.max(a1v, axis=1, keepdims=True)
        idx = jnp.min(
            jnp.where(a1v == m, a1i, jnp.float32(3.0e38))
        ).astype(jnp.int32)

        sx1 = x1s_ref[idx]
        sy1 = y1s_ref[idx]
        sx2 = x2s_ref[idx]
        sy2 = y2s_ref[idx]
        sarea = (sx2 - sx1) * (sy2 - sy1)

        xx1 = jnp.maximum(sx1, x1_ref[:])
        yy1 = jnp.maximum(sy1, y1_ref[:])
        xx2 = jnp.minimum(sx2, x2_ref[:])
        yy2 = jnp.minimum(sy2, y2_ref[:])
        inter = jnp.maximum(xx2 - xx1, 0.0) * jnp.maximum(yy2 - yy1, 0.0)
        iou = inter / (sarea + area_ref[:] - inter + jnp.float32(1e-9))
        s_new = jnp.where(iou > jnp.float32(_THRESH), jnp.float32(_NEG), s)

        detrow = jnp.where(
            lane == 0,
            sx1,
            jnp.where(
                lane == 1,
                sy1,
                jnp.where(lane == 2, sx2, jnp.where(lane == 3, sy2, m)),
            ),
        )
        out_ref[pl.ds(t, 1), :] = detrow
        return s_new

    def body(i, s):
        s = one_round(2 * i, s)
        return one_round(2 * i + 1, s)

    lax.fori_loop(0, _MAX_OUT // 2, body, s_ref[:])


def _run_nms(x1, y1, x2, y2, s, x1f, y1f, x2f, y2f):
    return pl.pallas_call(
        _nms_body,
        out_shape=jax.ShapeDtypeStruct((_MAX_OUT + 4, _COLS), jnp.float32),
        in_specs=[pl.BlockSpec(memory_space=pltpu.VMEM)] * 5
        + [pl.BlockSpec(memory_space=pltpu.SMEM)] * 4,
        out_specs=pl.BlockSpec(memory_space=pltpu.VMEM),
        scratch_shapes=[
            pltpu.VMEM((_ROWS, _COLS), jnp.float32),
        ],
    )(x1, y1, x2, y2, s, x1f, y1f, x2f, y2f)


def kernel(boxes, scores):
    zpad = jnp.zeros((_PAD,), jnp.float32)
    x1f = jnp.concatenate([boxes[:, 0], zpad])
    y1f = jnp.concatenate([boxes[:, 1], zpad])
    x2f = jnp.concatenate([boxes[:, 2], zpad])
    y2f = jnp.concatenate([boxes[:, 3], zpad])
    x1 = x1f.reshape(_ROWS, _COLS)
    y1 = y1f.reshape(_ROWS, _COLS)
    x2 = x2f.reshape(_ROWS, _COLS)
    y2 = y2f.reshape(_ROWS, _COLS)
    s = jnp.concatenate([scores, jnp.full((_PAD,), _PAD_SCORE)]).reshape(
        _ROWS, _COLS
    )
    out = _run_nms(x1, y1, x2, y2, s, x1f, y1f, x2f, y2f)
    return out[:_MAX_OUT, :5]


# speedup vs baseline: 31.2950x; 1.0080x over previous
import jax
import jax.numpy as jnp
from jax import lax
from jax.experimental import pallas as pl
from jax.experimental.pallas import tpu as pltpu

_N = 20000
_MAX_OUT = 300
_ROWS = 160
_COLS = 128
_PAD = _ROWS * _COLS - _N
_NEG = -1e9
_PAD_SCORE = -3.0e38
_THRESH = 0.7
_NBLK = _ROWS // 8


def _nms_body(
    x1_ref, y1_ref, x2_ref, y2_ref, s_ref,
    x1s_ref, y1s_ref, x2s_ref, y2s_ref,
    out_ref, area_ref,
):
    area_ref[:] = (x2_ref[:] - x1_ref[:]) * (y2_ref[:] - y1_ref[:])

    row_iota = lax.broadcasted_iota(jnp.int32, (_ROWS, _COLS), 0)
    col_iota = lax.broadcasted_iota(jnp.int32, (_ROWS, _COLS), 1)
    linf = (row_iota * _COLS + col_iota).astype(jnp.float32)
    lane = lax.broadcasted_iota(jnp.int32, (1, _COLS), 1)

    def one_round(t, s):
        def merge(cv, ci, nv, ni):
            take = (nv > cv) | ((nv == cv) & (ni < ci))
            return jnp.where(take, nv, cv), jnp.where(take, ni, ci)

        vals = [s[8 * i : 8 * i + 8] for i in range(_NBLK)]
        idxs = [linf[8 * i : 8 * i + 8] for i in range(_NBLK)]
        while len(vals) > 1:
            nv, ni = [], []
            for i in range(0, len(vals) - 1, 2):
                v, ix = merge(vals[i], idxs[i], vals[i + 1], idxs[i + 1])
                nv.append(v)
                ni.append(ix)
            if len(vals) % 2:
                nv.append(vals[-1])
                ni.append(idxs[-1])
            vals, idxs = nv, ni
        v8, i8 = vals[0], idxs[0]
        for sh in (4, 2, 1):
            vr = pltpu.roll(v8, 8 - sh, 0)
            ir = pltpu.roll(i8, 8 - sh, 0)
            v8, i8 = merge(v8, i8, vr, ir)
        a1v = v8[0:1, :]
        a1i = i8[0:1, :]

        m = jnp.max(a1v, axis=1, keepdims=True)
        idx = jnp.min(
            jnp.where(a1v == m, a1i, jnp.float32(3.0e38))
        ).astype(jnp.int32)

        sx1 = x1s_ref[idx]
        sy1 = y1s_ref[idx]
        sx2 = x2s_ref[idx]
        sy2 = y2s_ref[idx]
        sarea = (sx2 - sx1) * (sy2 - sy1)

        xx1 = jnp.maximum(sx1, x1_ref[:])
        yy1 = jnp.maximum(sy1, y1_ref[:])
        xx2 = jnp.minimum(sx2, x2_ref[:])
        yy2 = jnp.minimum(sy2, y2_ref[:])
        inter = jnp.maximum(xx2 - xx1, 0.0) * jnp.maximum(yy2 - yy1, 0.0)
        iou = inter / (sarea + area_ref[:] - inter + jnp.float32(1e-9))
        s_new = jnp.where(iou > jnp.float32(_THRESH), jnp.float32(_NEG), s)

        detrow = jnp.where(
            lane == 0,
            sx1,
            jnp.where(
                lane == 1,
                sy1,
                jnp.where(lane == 2, sx2, jnp.where(lane == 3, sy2, m)),
            ),
        )
        out_ref[pl.ds(t, 1), :] = detrow
        return s_new

    def body(i, s):
        s = one_round(4 * i, s)
        s = one_round(4 * i + 1, s)
        s = one_round(4 * i + 2, s)
        return one_round(4 * i + 3, s)

    lax.fori_loop(0, _MAX_OUT // 4, body, s_ref[:])


def _run_nms(x1, y1, x2, y2, s, x1f, y1f, x2f, y2f):
    return pl.pallas_call(
        _nms_body,
        out_shape=jax.ShapeDtypeStruct((_MAX_OUT + 4, _COLS), jnp.float32),
        in_specs=[pl.BlockSpec(memory_space=pltpu.VMEM)] * 5
        + [pl.BlockSpec(memory_space=pltpu.SMEM)] * 4,
        out_specs=pl.BlockSpec(memory_space=pltpu.VMEM),
        scratch_shapes=[
            pltpu.VMEM((_ROWS, _COLS), jnp.float32),
        ],
    )(x1, y1, x2, y2, s, x1f, y1f, x2f, y2f)


def kernel(boxes, scores):
    zpad = jnp.zeros((_PAD,), jnp.float32)
    x1f = jnp.concatenate([boxes[:, 0], zpad])
    y1f = jnp.concatenate([boxes[:, 1], zpad])
    x2f = jnp.concatenate([boxes[:, 2], zpad])
    y2f = jnp.concatenate([boxes[:, 3], zpad])
    x1 = x1f.reshape(_ROWS, _COLS)
    y1 = y1f.reshape(_ROWS, _COLS)
    x2 = x2f.reshape(_ROWS, _COLS)
    y2 = y2f.reshape(_ROWS, _COLS)
    s = jnp.concatenate([scores, jnp.full((_PAD,), _PAD_SCORE)]).reshape(
        _ROWS, _COLS
    )
    out = _run_nms(x1, y1, x2, y2, s, x1f, y1f, x2f, y2f)
    return out[:_MAX_OUT, :5]
